# baseline (speedup 1.0000x reference)
import functools

import jax
import jax.numpy as jnp
from jax import lax
from jax.experimental import pallas as pl
from jax.experimental.pallas import tpu as pltpu
from jax.experimental.pallas import tpu_sc as plsc

VOCAB = 100000
BATCH = 32
Q_LEN = 8
BB = 4


def _fill_body(ls_ref, ls_sq_ref, out_ref, base_ref, peaks_ref):
    i = pl.program_id(0)
    alpha = jax.nn.sigmoid(ls_ref[pl.ds(i * BB, BB), :])
    base = (1.0 - alpha) * jnp.float32(1.0 / VOCAB)
    log_base = jnp.maximum(jnp.log(base), jnp.float32(-1e6))
    out_ref[...] = jnp.broadcast_to(log_base[..., None], (BB, Q_LEN, VOCAB))

    @pl.when(i == 0)
    def _():
        a = jax.nn.sigmoid(ls_sq_ref[...])
        b = (1.0 - a) * jnp.float32(1.0 / VOCAB)
        base_ref[...] = jnp.maximum(jnp.log(b), jnp.float32(-1e6))
        peaks_ref[...] = jnp.maximum(jnp.log(b + a), jnp.float32(-1e6))


def _sc_scatter(base_hbm, peaks_hbm, ids_hbm, out_ref, base_v, peaks_v, ids_v, tmp, sem):
    s = lax.axis_index("s")
    d1 = pltpu.async_copy(base_hbm.at[s], base_v, sem)
    d2 = pltpu.async_copy(peaks_hbm.at[s], peaks_v, sem)
    d3 = pltpu.async_copy(ids_hbm.at[s], ids_v, sem)
    d1.wait()
    d2.wait()
    d3.wait()
    base_vec = base_v[...]
    peaks_vec = peaks_v[...]
    ids_vec = ids_v[...]
    lanes = lax.broadcasted_iota(jnp.int32, (16,), 0)
    descs = []
    for q in range(16):
        p = ids_vec[q]
        a16 = (p // 16) * 16
        vec = jnp.where(lanes == p - a16, peaks_vec[q], base_vec[q])
        tmp[pl.ds(q * 16, 16)] = vec
        descs.append(
            pltpu.async_copy(
                tmp.at[pl.ds(q * 16, 16)],
                out_ref.at[s * 2 + q // 8, q % 8, pl.ds(a16, 16)],
                sem,
            )
        )
    for d in descs:
        d.wait()


@jax.jit
def kernel(log_snr, input_ids):
    filled, base_rows, peak_rows = pl.pallas_call(
        _fill_body,
        grid=(BATCH // BB,),
        in_specs=[
            pl.BlockSpec((BATCH, Q_LEN), lambda i: (0, 0)),
            pl.BlockSpec((16, 16), lambda i: (0, 0)),
        ],
        out_specs=[
            pl.BlockSpec((BB, Q_LEN, VOCAB), lambda i: (i, 0, 0)),
            pl.BlockSpec((16, 16), lambda i: (0, 0)),
            pl.BlockSpec((16, 16), lambda i: (0, 0)),
        ],
        out_shape=[
            jax.ShapeDtypeStruct((BATCH, Q_LEN, VOCAB), jnp.float32),
            jax.ShapeDtypeStruct((16, 16), jnp.float32),
            jax.ShapeDtypeStruct((16, 16), jnp.float32),
        ],
    )(log_snr, log_snr.reshape(16, 16))

    out_ref = jax.new_ref(filled)
    mesh = plsc.VectorSubcoreMesh(core_axis_name="c", subcore_axis_name="s", num_cores=1)
    scatter = functools.partial(
        pl.kernel,
        mesh=mesh,
        scratch_types=[
            pltpu.VMEM((16,), jnp.float32),
            pltpu.VMEM((16,), jnp.float32),
            pltpu.VMEM((16,), jnp.int32),
            pltpu.VMEM((16 * 16,), jnp.float32),
            pltpu.SemaphoreType.DMA,
        ],
    )(_sc_scatter)
    scatter(base_rows, peak_rows, input_ids.astype(jnp.int32).reshape(16, 16), out_ref)
    return jax.freeze(out_ref)


# speedup vs baseline: 1.0307x; 1.0307x over previous
import functools

import jax
import jax.numpy as jnp
from jax import lax
from jax.experimental import pallas as pl
from jax.experimental.pallas import tpu as pltpu
from jax.experimental.pallas import tpu_sc as plsc

VOCAB = 100000
BATCH = 32
Q_LEN = 8
BB = 1


def _fill_body(ls_ref, ls_sq_ref, out_ref, base_ref, peaks_ref):
    i = pl.program_id(0)
    alpha = jax.nn.sigmoid(ls_ref[pl.ds(i * BB, BB), :])
    base = (1.0 - alpha) * jnp.float32(1.0 / VOCAB)
    log_base = jnp.maximum(jnp.log(base), jnp.float32(-1e6))
    out_ref[...] = jnp.broadcast_to(log_base[..., None], (BB, Q_LEN, VOCAB))

    @pl.when(i == 0)
    def _():
        a = jax.nn.sigmoid(ls_sq_ref[...])
        b = (1.0 - a) * jnp.float32(1.0 / VOCAB)
        base_ref[...] = jnp.maximum(jnp.log(b), jnp.float32(-1e6))
        peaks_ref[...] = jnp.maximum(jnp.log(b + a), jnp.float32(-1e6))


def _sc_scatter(base_hbm, peaks_hbm, ids_hbm, out_ref, base_v, peaks_v, ids_v, tmp, sem):
    s = lax.axis_index("s")
    d1 = pltpu.async_copy(base_hbm.at[s], base_v, sem)
    d2 = pltpu.async_copy(peaks_hbm.at[s], peaks_v, sem)
    d3 = pltpu.async_copy(ids_hbm.at[s], ids_v, sem)
    d1.wait()
    d2.wait()
    d3.wait()
    base_vec = base_v[...]
    peaks_vec = peaks_v[...]
    ids_vec = ids_v[...]
    lanes = lax.broadcasted_iota(jnp.int32, (16,), 0)
    descs = []
    for q in range(16):
        p = ids_vec[q]
        a16 = (p // 16) * 16
        vec = jnp.where(lanes == p - a16, peaks_vec[q], base_vec[q])
        tmp[pl.ds(q * 16, 16)] = vec
        descs.append(
            pltpu.async_copy(
                tmp.at[pl.ds(q * 16, 16)],
                out_ref.at[s * 2 + q // 8, q % 8, pl.ds(a16, 16)],
                sem,
            )
        )
    for d in descs:
        d.wait()


@jax.jit
def kernel(log_snr, input_ids):
    filled, base_rows, peak_rows = pl.pallas_call(
        _fill_body,
        grid=(BATCH // BB,),
        in_specs=[
            pl.BlockSpec((BATCH, Q_LEN), lambda i: (0, 0)),
            pl.BlockSpec((16, 16), lambda i: (0, 0)),
        ],
        out_specs=[
            pl.BlockSpec((BB, Q_LEN, VOCAB), lambda i: (i, 0, 0)),
            pl.BlockSpec((16, 16), lambda i: (0, 0)),
            pl.BlockSpec((16, 16), lambda i: (0, 0)),
        ],
        out_shape=[
            jax.ShapeDtypeStruct((BATCH, Q_LEN, VOCAB), jnp.float32),
            jax.ShapeDtypeStruct((16, 16), jnp.float32),
            jax.ShapeDtypeStruct((16, 16), jnp.float32),
        ],
    )(log_snr, log_snr.reshape(16, 16))

    out_ref = jax.new_ref(filled)
    mesh = plsc.VectorSubcoreMesh(core_axis_name="c", subcore_axis_name="s", num_cores=1)
    scatter = functools.partial(
        pl.kernel,
        mesh=mesh,
        scratch_types=[
            pltpu.VMEM((16,), jnp.float32),
            pltpu.VMEM((16,), jnp.float32),
            pltpu.VMEM((16,), jnp.int32),
            pltpu.VMEM((16 * 16,), jnp.float32),
            pltpu.SemaphoreType.DMA,
        ],
    )(_sc_scatter)
    scatter(base_rows, peak_rows, input_ids.astype(jnp.int32).reshape(16, 16), out_ref)
    return jax.freeze(out_ref)


# speedup vs baseline: 1.0441x; 1.0130x over previous
import functools

import jax
import jax.numpy as jnp
from jax import lax
from jax.experimental import pallas as pl
from jax.experimental.pallas import tpu as pltpu
from jax.experimental.pallas import tpu_sc as plsc

VOCAB = 100000
BATCH = 32
Q_LEN = 8
BB = 2


def _fill_body(ls_ref, ls_sq_ref, out_ref, base_ref, peaks_ref):
    i = pl.program_id(0)
    alpha = jax.nn.sigmoid(ls_ref[pl.ds(i * BB, BB), :])
    base = (1.0 - alpha) * jnp.float32(1.0 / VOCAB)
    log_base = jnp.maximum(jnp.log(base), jnp.float32(-1e6))
    out_ref[...] = jnp.broadcast_to(log_base[..., None], (BB, Q_LEN, VOCAB))

    @pl.when(i == 0)
    def _():
        a = jax.nn.sigmoid(ls_sq_ref[...])
        b = (1.0 - a) * jnp.float32(1.0 / VOCAB)
        base_ref[...] = jnp.maximum(jnp.log(b), jnp.float32(-1e6))
        peaks_ref[...] = jnp.maximum(jnp.log(b + a), jnp.float32(-1e6))


def _sc_scatter(base_hbm, peaks_hbm, ids_hbm, out_ref, base_v, peaks_v, ids_v, tmp, sem):
    s = lax.axis_index("s")
    d1 = pltpu.async_copy(base_hbm.at[s], base_v, sem)
    d2 = pltpu.async_copy(peaks_hbm.at[s], peaks_v, sem)
    d3 = pltpu.async_copy(ids_hbm.at[s], ids_v, sem)
    d1.wait()
    d2.wait()
    d3.wait()
    base_vec = base_v[...]
    peaks_vec = peaks_v[...]
    ids_vec = ids_v[...]
    lanes = lax.broadcasted_iota(jnp.int32, (16,), 0)
    descs = []
    for q in range(16):
        p = ids_vec[q]
        a16 = (p // 16) * 16
        vec = jnp.where(lanes == p - a16, peaks_vec[q], base_vec[q])
        tmp[pl.ds(q * 16, 16)] = vec
        descs.append(
            pltpu.async_copy(
                tmp.at[pl.ds(q * 16, 16)],
                out_ref.at[s * 2 + q // 8, q % 8, pl.ds(a16, 16)],
                sem,
            )
        )
    for d in descs:
        d.wait()


@jax.jit
def kernel(log_snr, input_ids):
    filled, base_rows, peak_rows = pl.pallas_call(
        _fill_body,
        grid=(BATCH // BB,),
        in_specs=[
            pl.BlockSpec((BATCH, Q_LEN), lambda i: (0, 0)),
            pl.BlockSpec((16, 16), lambda i: (0, 0)),
        ],
        out_specs=[
            pl.BlockSpec((BB, Q_LEN, VOCAB), lambda i: (i, 0, 0)),
            pl.BlockSpec((16, 16), lambda i: (0, 0)),
            pl.BlockSpec((16, 16), lambda i: (0, 0)),
        ],
        out_shape=[
            jax.ShapeDtypeStruct((BATCH, Q_LEN, VOCAB), jnp.float32),
            jax.ShapeDtypeStruct((16, 16), jnp.float32),
            jax.ShapeDtypeStruct((16, 16), jnp.float32),
        ],
    )(log_snr, log_snr.reshape(16, 16))

    out_ref = jax.new_ref(filled)
    mesh = plsc.VectorSubcoreMesh(core_axis_name="c", subcore_axis_name="s", num_cores=1)
    scatter = functools.partial(
        pl.kernel,
        mesh=mesh,
        scratch_types=[
            pltpu.VMEM((16,), jnp.float32),
            pltpu.VMEM((16,), jnp.float32),
            pltpu.VMEM((16,), jnp.int32),
            pltpu.VMEM((16 * 16,), jnp.float32),
            pltpu.SemaphoreType.DMA,
        ],
    )(_sc_scatter)
    scatter(base_rows, peak_rows, input_ids.astype(jnp.int32).reshape(16, 16), out_ref)
    return jax.freeze(out_ref)
